# baseline (speedup 1.0000x reference)
import functools

import jax
import jax.numpy as jnp
from jax.experimental import pallas as pl
from jax.experimental.pallas import tpu as pltpu
from jax.experimental.pallas import tpu_sc as plsc

E = 8
K = 2
D = 1024
F = 4096
TM = 512
TMG = 256
_NW = 32
_CH = 64


def _router_body(x_ref, wr_ref, br_ref, idx_ref, w_ref):
    x = x_ref[...]
    logits = jnp.dot(x, wr_ref[...], preferred_element_type=jnp.float32)
    logits = logits + br_ref[...]
    m = jnp.max(logits, axis=-1, keepdims=True)
    eg = jnp.exp(logits - m)
    gates = eg / jnp.sum(eg, axis=-1, keepdims=True)
    v1 = jnp.max(gates, axis=-1, keepdims=True)
    i1 = jnp.argmax(gates, axis=-1)[:, None]
    eiota = jax.lax.broadcasted_iota(jnp.int32, gates.shape, 1)
    masked = jnp.where(eiota == i1, -jnp.inf, gates)
    v2 = jnp.max(masked, axis=-1, keepdims=True)
    i2 = jnp.argmax(masked, axis=-1)[:, None]
    s = v1 + v2
    idx_ref[...] = jnp.concatenate([i1, i2], axis=1)
    w_ref[...] = jnp.concatenate([v1 / s, v2 / s], axis=1)


def _ffn_body(se_ref, sm_ref, slo_ref, shi_ref,
              x_ref, w1_ref, b1_ref, w2_ref, b2_ref,
              ys_ref, h_ref):
    s = pl.program_id(0)
    lo = slo_ref[s]
    hi = shi_ref[s]
    h = jnp.dot(x_ref[...].astype(jnp.bfloat16), w1_ref[0],
                preferred_element_type=jnp.float32)
    h_ref[...] = jnp.maximum(h + b1_ref[0], 0.0).astype(jnp.bfloat16)
    y = jnp.dot(h_ref[...], w2_ref[0],
                preferred_element_type=jnp.float32) + b2_ref[0]
    rows = jax.lax.broadcasted_iota(jnp.int32, (TMG, 1), 0)
    mask = (rows >= lo) & (rows < hi)
    ys_ref[...] = jnp.where(mask, y, ys_ref[...])


def _add_body(a_ref, b_ref, w_ref, o_ref):
    w = w_ref[...]
    o_ref[...] = a_ref[...] * w[:, 0:1] + b_ref[...] * w[:, 1:2]


def _sc_dispatch(xf, pos0, pos1):
    T, d = xf.shape
    tpw = T // _NW
    C = tpw // _CH
    mesh = plsc.VectorSubcoreMesh(core_axis_name="c", subcore_axis_name="s")

    @functools.partial(
        pl.kernel,
        out_type=jax.ShapeDtypeStruct((K * T, d), xf.dtype),
        mesh=mesh,
        scratch_types=[
            pltpu.VMEM((C, _CH), jnp.int32),
            pltpu.VMEM((C, _CH), jnp.int32),
            pltpu.VMEM((_CH, d), xf.dtype),
            pltpu.SemaphoreType.DMA,
        ],
    )
    def k(x_hbm, p0_hbm, p1_hbm, out_hbm, i0_v, i1_v, rows_v, sem):
        wid = jax.lax.axis_index("s") * 2 + jax.lax.axis_index("c")
        tb = wid * tpw
        pltpu.sync_copy(p0_hbm.at[wid], i0_v)
        pltpu.sync_copy(p1_hbm.at[wid], i1_v)

        @pl.loop(0, C)
        def _(ci):
            pltpu.sync_copy(x_hbm.at[pl.ds(tb + ci * _CH, _CH)], rows_v)
            pltpu.sync_copy(rows_v, out_hbm.at[i0_v.at[ci]])
            pltpu.sync_copy(rows_v, out_hbm.at[i1_v.at[ci]])

    return k(xf, pos0, pos1)


def _sc_gather(data, idx):
    n = idx.shape[0]
    d = data.shape[1]
    bpw = n // _NW
    mesh = plsc.VectorSubcoreMesh(core_axis_name="c", subcore_axis_name="s")

    @functools.partial(
        pl.kernel,
        out_type=jax.ShapeDtypeStruct((n, d), data.dtype),
        mesh=mesh,
        scratch_types=[
            pltpu.VMEM((bpw,), jnp.int32),
            pltpu.VMEM((_CH, d), data.dtype),
            pltpu.SemaphoreType.DMA,
        ],
    )
    def k(data_hbm, idx_hbm, out_hbm, idx_v, rows_v, sem):
        wid = jax.lax.axis_index("s") * 2 + jax.lax.axis_index("c")
        base = wid * bpw
        pltpu.sync_copy(idx_hbm.at[pl.ds(base, bpw)], idx_v)

        @pl.loop(0, bpw // _CH)
        def _(ci):
            off = ci * _CH
            pltpu.async_copy(data_hbm.at[idx_v.at[pl.ds(off, _CH)]],
                             rows_v, sem).wait()
            pltpu.sync_copy(rows_v, out_hbm.at[pl.ds(base + off, _CH)])

    return k(data, idx)


def kernel(x, Wr, br, W1, b1, W2, b2):
    orig_shape = x.shape
    xf = x.reshape(-1, D)
    T = xf.shape[0]
    P = K * T

    top_idx, top_w = pl.pallas_call(
        _router_body,
        grid=(T // TM,),
        in_specs=[
            pl.BlockSpec((TM, D), lambda m: (m, 0)),
            pl.BlockSpec((D, E), lambda m: (0, 0)),
            pl.BlockSpec((1, E), lambda m: (0, 0)),
        ],
        out_specs=[
            pl.BlockSpec((TM, K), lambda m: (m, 0)),
            pl.BlockSpec((TM, K), lambda m: (m, 0)),
        ],
        out_shape=[
            jax.ShapeDtypeStruct((T, K), jnp.int32),
            jax.ShapeDtypeStruct((T, K), jnp.float32),
        ],
    )(xf, Wr, br.reshape(1, E))

    ef = top_idx.reshape(-1)
    onehot = (ef[:, None] == jnp.arange(E, dtype=jnp.int32)[None, :])
    onehot = onehot.astype(jnp.int32)
    counts = jnp.sum(onehot, axis=0)
    offsets = jnp.concatenate(
        [jnp.zeros((1,), jnp.int32), jnp.cumsum(counts, dtype=jnp.int32)])
    rank = jnp.sum((jnp.cumsum(onehot, axis=0) - onehot) * onehot, axis=1)
    position = offsets[ef] + rank
    pos2 = position.reshape(T, K)
    pos0 = pos2[:, 0]
    pos1 = pos2[:, 1]

    NT = P // TMG
    NSLOT = NT + E - 1
    gs = offsets[:-1]
    ge = offsets[1:]
    first = gs // TMG
    last = jnp.where(ge > gs, (ge - 1) // TMG, first - 1)
    gtiles = jnp.maximum(last - first + 1, 0)
    slot_e = jnp.repeat(jnp.arange(E, dtype=jnp.int32), gtiles,
                        total_repeat_length=NSLOT)
    gstart = jnp.concatenate(
        [jnp.zeros((1,), jnp.int32), jnp.cumsum(gtiles, dtype=jnp.int32)])
    sidx = jnp.arange(NSLOT, dtype=jnp.int32)
    valid = sidx < gstart[E]
    slot_m = first[slot_e] + (sidx - gstart[slot_e])
    slot_m = jnp.where(valid, slot_m, NT - 1)
    slot_lo = jnp.where(valid, jnp.maximum(gs[slot_e] - slot_m * TMG, 0), 0)
    slot_hi = jnp.where(valid, jnp.minimum(ge[slot_e] - slot_m * TMG, TMG), 0)

    tpw = T // _NW
    xs = _sc_dispatch(xf,
                      pos0.reshape(_NW, tpw // _CH, _CH),
                      pos1.reshape(_NW, tpw // _CH, _CH))

    W1_bf = W1.astype(jnp.bfloat16)
    W2_bf = W2.astype(jnp.bfloat16)
    grid_spec = pltpu.PrefetchScalarGridSpec(
        num_scalar_prefetch=4,
        grid=(NSLOT,),
        in_specs=[
            pl.BlockSpec((TMG, D), lambda s, se, sm, slo, shi: (sm[s], 0)),
            pl.BlockSpec((1, D, F), lambda s, se, sm, slo, shi: (se[s], 0, 0)),
            pl.BlockSpec((1, 1, F), lambda s, se, sm, slo, shi: (se[s], 0, 0)),
            pl.BlockSpec((1, F, D), lambda s, se, sm, slo, shi: (se[s], 0, 0)),
            pl.BlockSpec((1, 1, D), lambda s, se, sm, slo, shi: (se[s], 0, 0)),
        ],
        out_specs=pl.BlockSpec((TMG, D), lambda s, se, sm, slo, shi: (sm[s], 0)),
        scratch_shapes=[pltpu.VMEM((TMG, F), jnp.bfloat16)],
    )
    ys = pl.pallas_call(
        _ffn_body,
        grid_spec=grid_spec,
        out_shape=jax.ShapeDtypeStruct((P, D), jnp.float32),
    )(slot_e, slot_m, slot_lo, slot_hi,
      xs, W1_bf, b1.reshape(E, 1, F), W2_bf, b2.reshape(E, 1, D))

    gidx = jnp.concatenate([pos0, pos1])
    g = _sc_gather(ys, gidx)

    out = pl.pallas_call(
        _add_body,
        grid=(T // TM,),
        in_specs=[
            pl.BlockSpec((TM, D), lambda m: (m, 0)),
            pl.BlockSpec((TM, D), lambda m: (m + T // TM, 0)),
            pl.BlockSpec((TM, K), lambda m: (m, 0)),
        ],
        out_specs=pl.BlockSpec((TM, D), lambda m: (m, 0)),
        out_shape=jax.ShapeDtypeStruct((T, D), jnp.float32),
    )(g, g, top_w)

    return out.reshape(orig_shape)


# speedup vs baseline: 1.0165x; 1.0165x over previous
import functools

import jax
import jax.numpy as jnp
from jax.experimental import pallas as pl
from jax.experimental.pallas import tpu as pltpu
from jax.experimental.pallas import tpu_sc as plsc

E = 8
K = 2
D = 1024
F = 4096
TM = 512
TMG = 512
_NW = 32
_CH = 64


def _router_body(x_ref, wr_ref, br_ref, idx_ref, w_ref):
    x = x_ref[...]
    logits = jnp.dot(x, wr_ref[...], preferred_element_type=jnp.float32)
    logits = logits + br_ref[...]
    m = jnp.max(logits, axis=-1, keepdims=True)
    eg = jnp.exp(logits - m)
    gates = eg / jnp.sum(eg, axis=-1, keepdims=True)
    v1 = jnp.max(gates, axis=-1, keepdims=True)
    i1 = jnp.argmax(gates, axis=-1)[:, None]
    eiota = jax.lax.broadcasted_iota(jnp.int32, gates.shape, 1)
    masked = jnp.where(eiota == i1, -jnp.inf, gates)
    v2 = jnp.max(masked, axis=-1, keepdims=True)
    i2 = jnp.argmax(masked, axis=-1)[:, None]
    s = v1 + v2
    idx_ref[...] = jnp.concatenate([i1, i2], axis=1)
    w_ref[...] = jnp.concatenate([v1 / s, v2 / s], axis=1)


def _ffn_body(se_ref, sm_ref, slo_ref, shi_ref,
              x_ref, w1_ref, b1_ref, w2_ref, b2_ref,
              ys_ref, h_ref):
    s = pl.program_id(0)
    lo = slo_ref[s]
    hi = shi_ref[s]
    h = jnp.dot(x_ref[...].astype(jnp.bfloat16), w1_ref[0],
                preferred_element_type=jnp.float32)
    h_ref[...] = jnp.maximum(h + b1_ref[0], 0.0).astype(jnp.bfloat16)
    y = jnp.dot(h_ref[...], w2_ref[0],
                preferred_element_type=jnp.float32) + b2_ref[0]
    rows = jax.lax.broadcasted_iota(jnp.int32, (TMG, 1), 0)
    mask = (rows >= lo) & (rows < hi)
    ys_ref[...] = jnp.where(mask, y, ys_ref[...])


def _meta_body(idx_ref, pos_ref, slots_ref, ohs_ref, s_ref):
    B = 512
    T = idx_ref.shape[0]
    NTl = (K * T) // TMG
    idx = idx_ref[...]
    e0 = idx[:, 0:1]
    e1 = idx[:, 1:2]
    eids = jax.lax.broadcasted_iota(jnp.int32, (T, E), 1)
    oh0 = (eids == e0).astype(jnp.float32)
    oh1 = (eids == e1).astype(jnp.float32)
    ohs_ref[...] = oh0 + oh1
    li = jax.lax.broadcasted_iota(jnp.int32, (B, B), 0)
    lj = jax.lax.broadcasted_iota(jnp.int32, (B, B), 1)
    Lt = (li > lj).astype(jnp.float32)

    def body(b, carry):
        blk = ohs_ref[pl.ds(b * B, B), :]
        s_loc = jnp.dot(Lt, blk, preferred_element_type=jnp.float32,
                        precision=jax.lax.Precision.HIGHEST)
        s_ref[pl.ds(b * B, B), :] = s_loc + carry
        return carry + jnp.sum(blk, axis=0, keepdims=True)

    counts = jax.lax.fori_loop(0, T // B, body,
                               jnp.zeros((1, E), jnp.float32))
    ui = jax.lax.broadcasted_iota(jnp.int32, (E, E), 0)
    uj = jax.lax.broadcasted_iota(jnp.int32, (E, E), 1)
    U = (ui < uj).astype(jnp.float32)
    offs = jnp.dot(counts, U, preferred_element_type=jnp.float32,
                   precision=jax.lax.Precision.HIGHEST)
    so = s_ref[...] + offs
    pos0 = jnp.sum(oh0 * so, axis=1, keepdims=True)
    pos1 = jnp.sum(oh1 * so, axis=1, keepdims=True)
    pos_ref[...] = jnp.concatenate([pos0, pos1], axis=1).astype(jnp.int32)

    gs = offs.astype(jnp.int32)
    ge = (offs + counts).astype(jnp.int32)
    first = gs // TMG
    last = jnp.where(ge > gs, (ge - 1) // TMG, first - 1)
    gtiles = jnp.maximum(last - first + 1, 0)
    gstart = jnp.dot(gtiles.astype(jnp.float32), U,
                     preferred_element_type=jnp.float32,
                     precision=jax.lax.Precision.HIGHEST).astype(jnp.int32)
    gtot = jnp.sum(gtiles, axis=1, keepdims=True)
    NS = slots_ref.shape[0]
    sidx = jax.lax.broadcasted_iota(jnp.int32, (NS, 1), 0)
    gsb = jnp.broadcast_to(gstart, (NS, E))
    slot_e = jnp.sum((gsb <= sidx).astype(jnp.int32), axis=1,
                     keepdims=True) - 1
    valid = sidx < gtot
    ohse = (jax.lax.broadcasted_iota(jnp.int32, (NS, E), 1)
            == slot_e).astype(jnp.float32)

    def sel(v):
        vb = jnp.broadcast_to(v, (NS, E)).astype(jnp.float32)
        return jnp.sum(ohse * vb, axis=1, keepdims=True).astype(jnp.int32)

    m = sel(first) + (sidx - sel(gstart))
    slot_m = jnp.where(valid, m, NTl - 1)
    slot_lo = jnp.where(valid, jnp.maximum(sel(gs) - m * TMG, 0), 0)
    slot_hi = jnp.where(valid, jnp.minimum(sel(ge) - m * TMG, TMG), 0)
    slots_ref[...] = jnp.concatenate(
        [slot_e, slot_m, slot_lo, slot_hi], axis=1)


def _add_body(a_ref, b_ref, w_ref, o_ref):
    w = w_ref[...]
    o_ref[...] = a_ref[...] * w[:, 0:1] + b_ref[...] * w[:, 1:2]


def _sc_dispatch(xf, pos0, pos1):
    T, d = xf.shape
    tpw = T // _NW
    C = tpw // _CH
    mesh = plsc.VectorSubcoreMesh(core_axis_name="c", subcore_axis_name="s")

    @functools.partial(
        pl.kernel,
        out_type=jax.ShapeDtypeStruct((K * T, d), xf.dtype),
        mesh=mesh,
        scratch_types=[
            pltpu.VMEM((C, _CH), jnp.int32),
            pltpu.VMEM((C, _CH), jnp.int32),
            pltpu.VMEM((_CH, d), xf.dtype),
            pltpu.SemaphoreType.DMA,
        ],
    )
    def k(x_hbm, p0_hbm, p1_hbm, out_hbm, i0_v, i1_v, rows_v, sem):
        wid = jax.lax.axis_index("s") * 2 + jax.lax.axis_index("c")
        tb = wid * tpw
        pltpu.sync_copy(p0_hbm.at[wid], i0_v)
        pltpu.sync_copy(p1_hbm.at[wid], i1_v)

        @pl.loop(0, C)
        def _(ci):
            pltpu.sync_copy(x_hbm.at[pl.ds(tb + ci * _CH, _CH)], rows_v)
            pltpu.sync_copy(rows_v, out_hbm.at[i0_v.at[ci]])
            pltpu.sync_copy(rows_v, out_hbm.at[i1_v.at[ci]])

    return k(xf, pos0, pos1)


def _sc_gather(data, idx):
    n = idx.shape[0]
    d = data.shape[1]
    bpw = n // _NW
    mesh = plsc.VectorSubcoreMesh(core_axis_name="c", subcore_axis_name="s")

    @functools.partial(
        pl.kernel,
        out_type=jax.ShapeDtypeStruct((n, d), data.dtype),
        mesh=mesh,
        scratch_types=[
            pltpu.VMEM((bpw,), jnp.int32),
            pltpu.VMEM((_CH, d), data.dtype),
            pltpu.SemaphoreType.DMA,
        ],
    )
    def k(data_hbm, idx_hbm, out_hbm, idx_v, rows_v, sem):
        wid = jax.lax.axis_index("s") * 2 + jax.lax.axis_index("c")
        base = wid * bpw
        pltpu.sync_copy(idx_hbm.at[pl.ds(base, bpw)], idx_v)

        @pl.loop(0, bpw // _CH)
        def _(ci):
            off = ci * _CH
            pltpu.async_copy(data_hbm.at[idx_v.at[pl.ds(off, _CH)]],
                             rows_v, sem).wait()
            pltpu.sync_copy(rows_v, out_hbm.at[pl.ds(base + off, _CH)])

    return k(data, idx)


def kernel(x, Wr, br, W1, b1, W2, b2):
    orig_shape = x.shape
    xf = x.reshape(-1, D)
    T = xf.shape[0]
    P = K * T

    top_idx, top_w = pl.pallas_call(
        _router_body,
        grid=(T // TM,),
        in_specs=[
            pl.BlockSpec((TM, D), lambda m: (m, 0)),
            pl.BlockSpec((D, E), lambda m: (0, 0)),
            pl.BlockSpec((1, E), lambda m: (0, 0)),
        ],
        out_specs=[
            pl.BlockSpec((TM, K), lambda m: (m, 0)),
            pl.BlockSpec((TM, K), lambda m: (m, 0)),
        ],
        out_shape=[
            jax.ShapeDtypeStruct((T, K), jnp.int32),
            jax.ShapeDtypeStruct((T, K), jnp.float32),
        ],
    )(xf, Wr, br.reshape(1, E))

    NT = P // TMG
    NSLOT = NT + E - 1
    pos2, slots = pl.pallas_call(
        _meta_body,
        out_shape=[
            jax.ShapeDtypeStruct((T, K), jnp.int32),
            jax.ShapeDtypeStruct((64, 4), jnp.int32),
        ],
        scratch_shapes=[
            pltpu.VMEM((T, E), jnp.float32),
            pltpu.VMEM((T, E), jnp.float32),
        ],
    )(top_idx)
    pos0 = pos2[:, 0]
    pos1 = pos2[:, 1]
    slot_e = slots[:NSLOT, 0]
    slot_m = slots[:NSLOT, 1]
    slot_lo = slots[:NSLOT, 2]
    slot_hi = slots[:NSLOT, 3]

    tpw = T // _NW
    xs = _sc_dispatch(xf,
                      pos0.reshape(_NW, tpw // _CH, _CH),
                      pos1.reshape(_NW, tpw // _CH, _CH))

    W1_bf = W1.astype(jnp.bfloat16)
    W2_bf = W2.astype(jnp.bfloat16)
    grid_spec = pltpu.PrefetchScalarGridSpec(
        num_scalar_prefetch=4,
        grid=(NSLOT,),
        in_specs=[
            pl.BlockSpec((TMG, D), lambda s, se, sm, slo, shi: (sm[s], 0)),
            pl.BlockSpec((1, D, F), lambda s, se, sm, slo, shi: (se[s], 0, 0)),
            pl.BlockSpec((1, 1, F), lambda s, se, sm, slo, shi: (se[s], 0, 0)),
            pl.BlockSpec((1, F, D), lambda s, se, sm, slo, shi: (se[s], 0, 0)),
            pl.BlockSpec((1, 1, D), lambda s, se, sm, slo, shi: (se[s], 0, 0)),
        ],
        out_specs=pl.BlockSpec((TMG, D), lambda s, se, sm, slo, shi: (sm[s], 0)),
        scratch_shapes=[pltpu.VMEM((TMG, F), jnp.bfloat16)],
    )
    ys = pl.pallas_call(
        _ffn_body,
        grid_spec=grid_spec,
        out_shape=jax.ShapeDtypeStruct((P, D), jnp.float32),
    )(slot_e, slot_m, slot_lo, slot_hi,
      xs, W1_bf, b1.reshape(E, 1, F), W2_bf, b2.reshape(E, 1, D))

    gidx = jnp.concatenate([pos0, pos1])
    g = _sc_gather(ys, gidx)

    out = pl.pallas_call(
        _add_body,
        grid=(T // TM,),
        in_specs=[
            pl.BlockSpec((TM, D), lambda m: (m, 0)),
            pl.BlockSpec((TM, D), lambda m: (m + T // TM, 0)),
            pl.BlockSpec((TM, K), lambda m: (m, 0)),
        ],
        out_specs=pl.BlockSpec((TM, D), lambda m: (m, 0)),
        out_shape=jax.ShapeDtypeStruct((T, D), jnp.float32),
    )(g, g, top_w)

    return out.reshape(orig_shape)


# speedup vs baseline: 1.0195x; 1.0029x over previous
import functools

import jax
import jax.numpy as jnp
from jax.experimental import pallas as pl
from jax.experimental.pallas import tpu as pltpu
from jax.experimental.pallas import tpu_sc as plsc

E = 8
K = 2
D = 1024
F = 4096
TM = 512
TMG = 512
_NW = 32
_CH = 32


def _router_body(x_ref, wr_ref, br_ref, idx_ref, w_ref):
    x = x_ref[...]
    logits = jnp.dot(x, wr_ref[...], preferred_element_type=jnp.float32)
    logits = logits + br_ref[...]
    m = jnp.max(logits, axis=-1, keepdims=True)
    eg = jnp.exp(logits - m)
    gates = eg / jnp.sum(eg, axis=-1, keepdims=True)
    v1 = jnp.max(gates, axis=-1, keepdims=True)
    i1 = jnp.argmax(gates, axis=-1)[:, None]
    eiota = jax.lax.broadcasted_iota(jnp.int32, gates.shape, 1)
    masked = jnp.where(eiota == i1, -jnp.inf, gates)
    v2 = jnp.max(masked, axis=-1, keepdims=True)
    i2 = jnp.argmax(masked, axis=-1)[:, None]
    s = v1 + v2
    idx_ref[...] = jnp.concatenate([i1, i2], axis=1)
    w_ref[...] = jnp.concatenate([v1 / s, v2 / s], axis=1)


def _ffn_body(se_ref, sm_ref, slo_ref, shi_ref,
              x_ref, w1_ref, b1_ref, w2_ref, b2_ref,
              ys_ref, h_ref):
    s = pl.program_id(0)
    lo = slo_ref[s]
    hi = shi_ref[s]
    h = jnp.dot(x_ref[...].astype(jnp.bfloat16), w1_ref[0],
                preferred_element_type=jnp.float32)
    h_ref[...] = jnp.maximum(h + b1_ref[0], 0.0).astype(jnp.bfloat16)
    y = jnp.dot(h_ref[...], w2_ref[0],
                preferred_element_type=jnp.float32) + b2_ref[0]
    rows = jax.lax.broadcasted_iota(jnp.int32, (TMG, 1), 0)
    mask = (rows >= lo) & (rows < hi)
    ys_ref[...] = jnp.where(mask, y, ys_ref[...])


def _meta_body(idx_ref, pos_ref, slots_ref, ohs_ref, s_ref):
    B = 512
    T = idx_ref.shape[0]
    NTl = (K * T) // TMG
    idx = idx_ref[...]
    e0 = idx[:, 0:1]
    e1 = idx[:, 1:2]
    eids = jax.lax.broadcasted_iota(jnp.int32, (T, E), 1)
    oh0 = (eids == e0).astype(jnp.float32)
    oh1 = (eids == e1).astype(jnp.float32)
    ohs_ref[...] = oh0 + oh1
    li = jax.lax.broadcasted_iota(jnp.int32, (B, B), 0)
    lj = jax.lax.broadcasted_iota(jnp.int32, (B, B), 1)
    Lt = (li > lj).astype(jnp.float32)

    def body(b, carry):
        blk = ohs_ref[pl.ds(b * B, B), :]
        s_loc = jnp.dot(Lt, blk, preferred_element_type=jnp.float32,
                        precision=jax.lax.Precision.HIGHEST)
        s_ref[pl.ds(b * B, B), :] = s_loc + carry
        return carry + jnp.sum(blk, axis=0, keepdims=True)

    counts = jax.lax.fori_loop(0, T // B, body,
                               jnp.zeros((1, E), jnp.float32))
    ui = jax.lax.broadcasted_iota(jnp.int32, (E, E), 0)
    uj = jax.lax.broadcasted_iota(jnp.int32, (E, E), 1)
    U = (ui < uj).astype(jnp.float32)
    offs = jnp.dot(counts, U, preferred_element_type=jnp.float32,
                   precision=jax.lax.Precision.HIGHEST)
    so = s_ref[...] + offs
    pos0 = jnp.sum(oh0 * so, axis=1, keepdims=True)
    pos1 = jnp.sum(oh1 * so, axis=1, keepdims=True)
    pos_ref[...] = jnp.concatenate([pos0, pos1], axis=1).astype(jnp.int32)

    gs = offs.astype(jnp.int32)
    ge = (offs + counts).astype(jnp.int32)
    first = gs // TMG
    last = jnp.where(ge > gs, (ge - 1) // TMG, first - 1)
    gtiles = jnp.maximum(last - first + 1, 0)
    gstart = jnp.dot(gtiles.astype(jnp.float32), U,
                     preferred_element_type=jnp.float32,
                     precision=jax.lax.Precision.HIGHEST).astype(jnp.int32)
    gtot = jnp.sum(gtiles, axis=1, keepdims=True)
    NS = slots_ref.shape[0]
    sidx = jax.lax.broadcasted_iota(jnp.int32, (NS, 1), 0)
    gsb = jnp.broadcast_to(gstart, (NS, E))
    slot_e = jnp.sum((gsb <= sidx).astype(jnp.int32), axis=1,
                     keepdims=True) - 1
    valid = sidx < gtot
    ohse = (jax.lax.broadcasted_iota(jnp.int32, (NS, E), 1)
            == slot_e).astype(jnp.float32)

    def sel(v):
        vb = jnp.broadcast_to(v, (NS, E)).astype(jnp.float32)
        return jnp.sum(ohse * vb, axis=1, keepdims=True).astype(jnp.int32)

    m = sel(first) + (sidx - sel(gstart))
    slot_m = jnp.where(valid, m, NTl - 1)
    slot_lo = jnp.where(valid, jnp.maximum(sel(gs) - m * TMG, 0), 0)
    slot_hi = jnp.where(valid, jnp.minimum(sel(ge) - m * TMG, TMG), 0)
    slots_ref[...] = jnp.concatenate(
        [slot_e, slot_m, slot_lo, slot_hi], axis=1)


def _add_body(a_ref, b_ref, w_ref, o_ref):
    w = w_ref[...]
    o_ref[...] = a_ref[...] * w[:, 0:1] + b_ref[...] * w[:, 1:2]


def _sc_dispatch(xf, pos0, pos1):
    T, d = xf.shape
    tpw = T // _NW
    C = tpw // _CH
    mesh = plsc.VectorSubcoreMesh(core_axis_name="c", subcore_axis_name="s")

    @functools.partial(
        pl.kernel,
        out_type=jax.ShapeDtypeStruct((K * T, d), xf.dtype),
        mesh=mesh,
        scratch_types=[
            pltpu.VMEM((C, _CH), jnp.int32),
            pltpu.VMEM((C, _CH), jnp.int32),
            pltpu.VMEM((_CH, d), xf.dtype),
            pltpu.VMEM((_CH, d), xf.dtype),
            pltpu.SemaphoreType.DMA,
            pltpu.SemaphoreType.DMA,
            pltpu.SemaphoreType.DMA,
            pltpu.SemaphoreType.DMA,
        ],
    )
    def k(x_hbm, p0_hbm, p1_hbm, out_hbm, i0_v, i1_v, ra, rb,
          sr0, sr1, sw0, sw1):
        wid = jax.lax.axis_index("s") * 2 + jax.lax.axis_index("c")
        tb = wid * tpw
        pltpu.sync_copy(p0_hbm.at[wid], i0_v)
        pltpu.sync_copy(p1_hbm.at[wid], i1_v)

        bufs = (ra, rb)
        rsems = (sr0, sr1)
        rh = [None] * C
        rh[0] = pltpu.async_copy(x_hbm.at[pl.ds(tb, _CH)], ra, sr0)
        if C > 1:
            rh[1] = pltpu.async_copy(x_hbm.at[pl.ds(tb + _CH, _CH)], rb, sr1)
        tail = []
        for ci in range(C):
            buf = bufs[ci % 2]
            rh[ci].wait()
            h0 = pltpu.async_copy(buf, out_hbm.at[i0_v.at[ci]], sw0)
            h1 = pltpu.async_copy(buf, out_hbm.at[i1_v.at[ci]], sw1)
            if ci + 2 < C:
                h0.wait()
                h1.wait()
                rh[ci + 2] = pltpu.async_copy(
                    x_hbm.at[pl.ds(tb + (ci + 2) * _CH, _CH)],
                    buf, rsems[ci % 2])
            else:
                tail.append(h0)
                tail.append(h1)
        for h in tail:
            h.wait()

    return k(xf, pos0, pos1)


def _sc_gather(data, idx):
    n = idx.shape[0]
    d = data.shape[1]
    bpw = n // _NW
    mesh = plsc.VectorSubcoreMesh(core_axis_name="c", subcore_axis_name="s")

    C = bpw // _CH

    @functools.partial(
        pl.kernel,
        out_type=jax.ShapeDtypeStruct((n, d), data.dtype),
        mesh=mesh,
        scratch_types=[
            pltpu.VMEM((bpw,), jnp.int32),
            pltpu.VMEM((_CH, d), data.dtype),
            pltpu.VMEM((_CH, d), data.dtype),
            pltpu.SemaphoreType.DMA,
            pltpu.SemaphoreType.DMA,
            pltpu.SemaphoreType.DMA,
            pltpu.SemaphoreType.DMA,
        ],
    )
    def k(data_hbm, idx_hbm, out_hbm, idx_v, ra, rb, sr0, sr1, sw0, sw1):
        wid = jax.lax.axis_index("s") * 2 + jax.lax.axis_index("c")
        base = wid * bpw
        pltpu.sync_copy(idx_hbm.at[pl.ds(base, bpw)], idx_v)

        bufs = (ra, rb)
        rsems = (sr0, sr1)
        wsems = (sw0, sw1)
        rh = [None] * C
        rh[0] = pltpu.async_copy(data_hbm.at[idx_v.at[pl.ds(0, _CH)]],
                                 ra, sr0)
        if C > 1:
            rh[1] = pltpu.async_copy(data_hbm.at[idx_v.at[pl.ds(_CH, _CH)]],
                                     rb, sr1)
        tail = []
        for ci in range(C):
            buf = bufs[ci % 2]
            rh[ci].wait()
            h = pltpu.async_copy(buf, out_hbm.at[pl.ds(base + ci * _CH, _CH)],
                                 wsems[ci % 2])
            if ci + 2 < C:
                h.wait()
                rh[ci + 2] = pltpu.async_copy(
                    data_hbm.at[idx_v.at[pl.ds((ci + 2) * _CH, _CH)]],
                    buf, rsems[ci % 2])
            else:
                tail.append(h)
        for h in tail:
            h.wait()

    return k(data, idx)


def kernel(x, Wr, br, W1, b1, W2, b2):
    orig_shape = x.shape
    xf = x.reshape(-1, D)
    T = xf.shape[0]
    P = K * T

    top_idx, top_w = pl.pallas_call(
        _router_body,
        grid=(T // TM,),
        in_specs=[
            pl.BlockSpec((TM, D), lambda m: (m, 0)),
            pl.BlockSpec((D, E), lambda m: (0, 0)),
            pl.BlockSpec((1, E), lambda m: (0, 0)),
        ],
        out_specs=[
            pl.BlockSpec((TM, K), lambda m: (m, 0)),
            pl.BlockSpec((TM, K), lambda m: (m, 0)),
        ],
        out_shape=[
            jax.ShapeDtypeStruct((T, K), jnp.int32),
            jax.ShapeDtypeStruct((T, K), jnp.float32),
        ],
    )(xf, Wr, br.reshape(1, E))

    NT = P // TMG
    NSLOT = NT + E - 1
    pos2, slots = pl.pallas_call(
        _meta_body,
        out_shape=[
            jax.ShapeDtypeStruct((T, K), jnp.int32),
            jax.ShapeDtypeStruct((64, 4), jnp.int32),
        ],
        scratch_shapes=[
            pltpu.VMEM((T, E), jnp.float32),
            pltpu.VMEM((T, E), jnp.float32),
        ],
    )(top_idx)
    pos0 = pos2[:, 0]
    pos1 = pos2[:, 1]
    slot_e = slots[:NSLOT, 0]
    slot_m = slots[:NSLOT, 1]
    slot_lo = slots[:NSLOT, 2]
    slot_hi = slots[:NSLOT, 3]

    tpw = T // _NW
    xs = _sc_dispatch(xf,
                      pos0.reshape(_NW, tpw // _CH, _CH),
                      pos1.reshape(_NW, tpw // _CH, _CH))

    W1_bf = W1.astype(jnp.bfloat16)
    W2_bf = W2.astype(jnp.bfloat16)
    grid_spec = pltpu.PrefetchScalarGridSpec(
        num_scalar_prefetch=4,
        grid=(NSLOT,),
        in_specs=[
            pl.BlockSpec((TMG, D), lambda s, se, sm, slo, shi: (sm[s], 0)),
            pl.BlockSpec((1, D, F), lambda s, se, sm, slo, shi: (se[s], 0, 0)),
            pl.BlockSpec((1, 1, F), lambda s, se, sm, slo, shi: (se[s], 0, 0)),
            pl.BlockSpec((1, F, D), lambda s, se, sm, slo, shi: (se[s], 0, 0)),
            pl.BlockSpec((1, 1, D), lambda s, se, sm, slo, shi: (se[s], 0, 0)),
        ],
        out_specs=pl.BlockSpec((TMG, D), lambda s, se, sm, slo, shi: (sm[s], 0)),
        scratch_shapes=[pltpu.VMEM((TMG, F), jnp.bfloat16)],
    )
    ys = pl.pallas_call(
        _ffn_body,
        grid_spec=grid_spec,
        out_shape=jax.ShapeDtypeStruct((P, D), jnp.float32),
    )(slot_e, slot_m, slot_lo, slot_hi,
      xs, W1_bf, b1.reshape(E, 1, F), W2_bf, b2.reshape(E, 1, D))

    gidx = jnp.concatenate([pos0, pos1])
    g = _sc_gather(ys, gidx)

    out = pl.pallas_call(
        _add_body,
        grid=(T // TM,),
        in_specs=[
            pl.BlockSpec((TM, D), lambda m: (m, 0)),
            pl.BlockSpec((TM, D), lambda m: (m + T // TM, 0)),
            pl.BlockSpec((TM, K), lambda m: (m, 0)),
        ],
        out_specs=pl.BlockSpec((TM, D), lambda m: (m, 0)),
        out_shape=jax.ShapeDtypeStruct((T, D), jnp.float32),
    )(g, g, top_w)

    return out.reshape(orig_shape)
